# Initial kernel scaffold; baseline (speedup 1.0000x reference)
import jax
import jax.numpy as jnp
from jax.experimental import pallas as pl


def kernel(data, position_embed, edge_index, ws_w, ws_b, d1w0, d1b0, d1w1, d1b1, d1w2, d1b2, d2w0, d2b0, d2w1, d2b1, d2w2, d2b2, d3w0, d3b0, d3w1, d3b1, d3w2, d3b2, fc3_w, fc3_b, gat_wl, gat_bl, gat_wr, gat_br, gat_att, gat_b, fc2_w, fc2_b, fc1_w, fc1_b):
    raise NotImplementedError("write your pallas kernel here")


# speedup vs baseline: 64.0822x; 64.0822x over previous
import functools

import jax
import jax.numpy as jnp
import numpy as np
from jax import lax
from jax.experimental import pallas as pl
from jax.experimental.pallas import tpu as pltpu

B, T, N, H, C = 16, 64, 62, 2, 256
POS = 16
E = 4000
R = B * N


def _branch_taps(w0, w1, w2):
    L1 = [w0[:, :, 0, 1], w0[:, :, 0, 0]]
    L2 = [w1[:, :, 0, 1], w1[:, :, 0, 0]]
    L3 = [w2[:, :, 0, 1], w2[:, :, 0, 0]]
    A = []
    for t in range(8):
        a, b, c = (t >> 2) & 1, (t >> 1) & 1, t & 1
        A.append(L3[a] @ L2[b] @ L1[c])
    return jnp.stack(A), L2, L3


def _branch_bias(A, L2, L3, b0, b1, b2, ws_b):
    w = np.arange(80)
    beta = jnp.zeros((80, 16), jnp.float32)
    for a in (0, 1):
        for b in (0, 1):
            ind = (w >= 4 * a + 2 * b).astype(np.float32)
            beta = beta + jnp.asarray(ind)[:, None] * (L3[a] @ (L2[b] @ b0))[None, :]
        ind = (w >= 4 * a).astype(np.float32)
        beta = beta + jnp.asarray(ind)[:, None] * (L3[a] @ b1)[None, :]
    beta = beta + b2[None, :]
    Awsb = A @ ws_b
    for t in range(8):
        ind = ((w - t >= 0) & (w - t < 64)).astype(np.float32)
        beta = beta + jnp.asarray(ind)[:, None] * Awsb[t][None, :]
    return beta


def _front_consts(ws_w, ws_b, convs):
    A_all, v_all, betas = [], [], []
    for (w0, b0, w1, b1, w2, b2) in convs:
        A, L2, L3 = _branch_taps(w0, w1, w2)
        A_all.append(A)
        v_all.append(A @ ws_w[:, 0])
        betas.append(_branch_bias(A, L2, L3, b0, b1, b2, ws_b))
    A_st = jnp.stack(A_all, axis=1)
    v_st = jnp.stack(v_all, axis=1)
    beta = jnp.stack(betas, axis=0)

    eye64 = np.stack([np.eye(64, k=t, dtype=np.float32) for t in range(8)])
    T_node = jnp.einsum('twv,tjc->wjcv', jnp.asarray(eye64), v_st)
    T_node = T_node.reshape(64, 3 * 16 * 64)
    bias_node = jnp.transpose(beta[:, :64, :], (0, 2, 1)).reshape(1, 3 * 16 * 64)

    eye16 = np.stack([np.eye(16, k=t, dtype=np.float32) for t in range(8)])
    G_p = jnp.einsum('tuw,tjcp->pujcw', jnp.asarray(eye16), A_st)
    G_p = G_p.reshape(512, 3 * 16 * 16)
    I2 = np.zeros((8, 8, 16), np.float32)
    for t in range(1, 8):
        for m in range(8):
            wp = m - 8 + t
            if 0 <= wp < 16:
                I2[t, m, wp] = 1.0
    G_s = jnp.einsum('tmw,tjc->mjcw', jnp.asarray(I2), v_st).reshape(8, 3 * 16 * 16)
    G_pos = jnp.concatenate([G_p, G_s], axis=0)
    bias_pos = jnp.transpose(beta[:, 64:, :], (0, 2, 1)).reshape(1, 3 * 16 * 16)
    return T_node, bias_node, G_pos, bias_pos


def _front_body(s_ref, p_ref, T_ref, bn_ref, G_ref, bp_ref, F_ref, bf_ref,
                wl_ref, bl_ref, wr_ref, br_ref, xl_ref, xr_ref, pos_ref):
    s = s_ref[...]
    out_node = jnp.dot(s, T_ref[...], preferred_element_type=jnp.float32)
    out_node = out_node + bn_ref[...]
    U = jnp.concatenate([p_ref[...], s[:, 56:64]], axis=1)
    out_pos = jnp.dot(U, G_ref[...], preferred_element_type=jnp.float32)
    out_pos = out_pos + bp_ref[...]

    def glu(o):
        k = o.shape[1] // 3
        return jnp.maximum(
            jnp.tanh(o[:, :k]) * jax.nn.sigmoid(o[:, k:2 * k]) + o[:, 2 * k:], 0.0)

    g_node = glu(out_node)
    g_pos = glu(out_pos)
    xl_ref[...] = jnp.dot(g_node, wl_ref[...],
                          preferred_element_type=jnp.float32) + bl_ref[...]
    xr_ref[...] = jnp.dot(g_node, wr_ref[...],
                          preferred_element_type=jnp.float32) + br_ref[...]
    pos_ref[...] = jnp.dot(g_pos, F_ref[...],
                           preferred_element_type=jnp.float32) + bf_ref[...]


def _mbuild_body(dst_ref, src_ref, m_ref):
    jt = lax.broadcasted_iota(jnp.int32, (N, E), 0)
    D = (jt == dst_ref[...]).astype(jnp.float32)
    it = lax.broadcasted_iota(jnp.int32, (E, N), 1)
    S = (it == src_ref[...]).astype(jnp.float32)
    m_ref[...] = jnp.dot(D, S, preferred_element_type=jnp.float32)


def _gat_body(xl_ref, xr_ref, m_ref, att_ref, w1_ref, w2m_ref, cv_ref, pre_ref):
    r = lax.broadcasted_iota(jnp.int32, (N, N), 0)
    c = lax.broadcasted_iota(jnp.int32, (N, N), 1)
    M = m_ref[...] + (r == c).astype(jnp.float32)
    mask = M > 0.0
    outs = []
    for h in range(H):
        Xl = xl_ref[0, :, h * C:(h + 1) * C]
        Xr = xr_ref[0, :, h * C:(h + 1) * C]
        att = att_ref[...][h][None, None, :]
        Z = Xl[None, :, :] + Xr[:, None, :]
        Z = jnp.where(Z >= 0.0, Z, 0.2 * Z)
        S = jnp.sum(Z * att, axis=-1)
        Sm = jnp.where(mask, S, -1e30)
        mx = jnp.max(Sm, axis=1, keepdims=True)
        P = jnp.exp(Sm - mx) * M
        den = jnp.sum(P, axis=1, keepdims=True)
        Aw = P / (den + 1e-16)
        outs.append(jnp.dot(Aw, Xl, preferred_element_type=jnp.float32))
    out = jnp.concatenate(outs, axis=1)
    q = jnp.sum(out * w1_ref[...], axis=0, keepdims=True)
    acc = jnp.dot(q, w2m_ref[...], preferred_element_type=jnp.float32)
    pre_ref[...] = jax.nn.sigmoid(acc + cv_ref[...])[None]


def kernel(data, position_embed, edge_index, ws_w, ws_b,
           d1w0, d1b0, d1w1, d1b1, d1w2, d1b2,
           d2w0, d2b0, d2w1, d2b1, d2w2, d2b2,
           d3w0, d3b0, d3w1, d3b1, d3w2, d3b2,
           fc3_w, fc3_b, gat_wl, gat_bl, gat_wr, gat_br, gat_att, gat_b,
           fc2_w, fc2_b, fc1_w, fc1_b):
    data = data.astype(jnp.float32)
    train = data[:, :62, :]
    target = data[:, 62:63, :]
    s = train.reshape(R, T)
    p = jnp.transpose(position_embed, (0, 2, 1, 3)).reshape(R, 32 * POS)

    convs = [(d1w0, d1b0, d1w1, d1b1, d1w2, d1b2),
             (d2w0, d2b0, d2w1, d2b1, d2w2, d2b2),
             (d3w0, d3b0, d3w1, d3b1, d3w2, d3b2)]
    T_node, bias_node, G_pos, bias_pos = _front_consts(ws_w, ws_b, convs)

    F = jnp.einsum('oc,wv->cwov', fc3_w, jnp.eye(POS, dtype=jnp.float32))
    F = F.reshape(16 * POS, 32 * POS)
    bias_f = jnp.repeat(fc3_b, POS)[None, :]

    xl, xr, pos_out = pl.pallas_call(
        _front_body,
        out_shape=[jax.ShapeDtypeStruct((R, H * C), jnp.float32),
                   jax.ShapeDtypeStruct((R, H * C), jnp.float32),
                   jax.ShapeDtypeStruct((R, 32 * POS), jnp.float32)],
    )(s, p, T_node, bias_node, G_pos, bias_pos, F, bias_f,
      gat_wl, gat_bl[None, :], gat_wr, gat_br[None, :])

    dst = edge_index[1].astype(jnp.int32)[None, :]
    src = edge_index[0].astype(jnp.int32)[:, None]
    Mcnt = pl.pallas_call(
        _mbuild_body,
        out_shape=jax.ShapeDtypeStruct((N, N), jnp.float32),
    )(dst, src)

    w1col = fc1_w[0][:, None]
    w1sum = jnp.sum(fc1_w)
    W2mat = jnp.einsum('k,tv->ktv', fc2_w[0],
                       jnp.eye(T, dtype=jnp.float32)).reshape(8 * T, T)
    gb = jnp.sum(gat_b.reshape(8, T) * fc2_w[0][:, None], axis=0)
    constv = (fc1_b[0] + w1sum * fc2_b[0] + w1sum * gb)[None, :]

    pre = pl.pallas_call(
        _gat_body,
        grid=(B,),
        in_specs=[
            pl.BlockSpec((1, N, H * C), lambda b: (b, 0, 0)),
            pl.BlockSpec((1, N, H * C), lambda b: (b, 0, 0)),
            pl.BlockSpec((N, N), lambda b: (0, 0)),
            pl.BlockSpec((H, C), lambda b: (0, 0)),
            pl.BlockSpec((N, 1), lambda b: (0, 0)),
            pl.BlockSpec((8 * T, T), lambda b: (0, 0)),
            pl.BlockSpec((1, T), lambda b: (0, 0)),
        ],
        out_specs=pl.BlockSpec((1, 1, T), lambda b: (b, 0, 0)),
        out_shape=jax.ShapeDtypeStruct((B, 1, T), jnp.float32),
    )(xl.reshape(B, N, H * C), xr.reshape(B, N, H * C), Mcnt,
      gat_att, w1col, W2mat, constv)

    pos_learned = jnp.transpose(pos_out.reshape(B, N, 32, POS), (0, 2, 1, 3))
    return (pre, target, pos_learned)


# speedup vs baseline: 65.2412x; 1.0181x over previous
import functools

import jax
import jax.numpy as jnp
import numpy as np
from jax import lax
from jax.experimental import pallas as pl
from jax.experimental.pallas import tpu as pltpu

B, T, N, H, C = 16, 64, 62, 2, 256
POS = 16
E = 4000
R = B * N


def _front_consts(ws_w, ws_b, convs):
    L1 = jnp.stack([jnp.stack([w0[:, :, 0, 1], w0[:, :, 0, 0]])
                    for (w0, _, _, _, _, _) in convs])
    L2 = jnp.stack([jnp.stack([w1[:, :, 0, 1], w1[:, :, 0, 0]])
                    for (_, _, w1, _, _, _) in convs])
    L3 = jnp.stack([jnp.stack([w2[:, :, 0, 1], w2[:, :, 0, 0]])
                    for (_, _, _, _, w2, _) in convs])
    B0 = jnp.stack([b for (_, b, _, _, _, _) in convs])
    B1 = jnp.stack([b for (_, _, _, b, _, _) in convs])
    B2 = jnp.stack([b for (_, _, _, _, _, b) in convs])

    P21 = jnp.einsum('jbik,jckl->jbcil', L2, L1)
    A6 = jnp.einsum('jaik,jbckl->jabcil', L3, P21)
    A_st = jnp.transpose(A6.reshape(3, 8, 16, 32), (1, 0, 2, 3))
    v_st = jnp.einsum('tjcp,p->tjc', A_st, ws_w[:, 0])

    w = np.arange(80)
    ind_ab = np.stack([(w >= 4 * a + 2 * b).astype(np.float32)
                       for a in (0, 1) for b in (0, 1)])
    ind_a = np.stack([(w >= 4 * a).astype(np.float32) for a in (0, 1)])
    ind_t = np.stack([((w - t >= 0) & (w - t < 64)).astype(np.float32)
                      for t in range(8)])
    C_ab = jnp.einsum('jaik,jbkl,jl->jabi', L3, L2, B0).reshape(3, 4, 16)
    C_a = jnp.einsum('jaik,jk->jai', L3, B1)
    Awsb = jnp.einsum('tjcp,p->tjc', A_st, ws_b)
    beta = (jnp.einsum('mw,jmi->jwi', jnp.asarray(ind_ab), C_ab)
            + jnp.einsum('aw,jai->jwi', jnp.asarray(ind_a), C_a)
            + B2[:, None, :]
            + jnp.einsum('tw,tjc->jwc', jnp.asarray(ind_t), Awsb))

    eye64 = np.stack([np.eye(64, k=t, dtype=np.float32) for t in range(8)])
    T_node = jnp.einsum('twv,tjc->wjcv', jnp.asarray(eye64), v_st)
    T_node = T_node.reshape(64, 3 * 16 * 64)
    bias_node = jnp.transpose(beta[:, :64, :], (0, 2, 1)).reshape(1, 3 * 16 * 64)

    eye16 = np.stack([np.eye(16, k=t, dtype=np.float32) for t in range(8)])
    G_p = jnp.einsum('tuw,tjcp->pujcw', jnp.asarray(eye16), A_st)
    G_p = G_p.reshape(512, 3 * 16 * 16)
    I2 = np.zeros((8, 8, 16), np.float32)
    for t in range(1, 8):
        for m in range(8):
            wp = m - 8 + t
            if 0 <= wp < 16:
                I2[t, m, wp] = 1.0
    G_s = jnp.einsum('tmw,tjc->mjcw', jnp.asarray(I2), v_st).reshape(8, 3 * 16 * 16)
    G_pos = jnp.concatenate([G_p, G_s], axis=0)
    bias_pos = jnp.transpose(beta[:, 64:, :], (0, 2, 1)).reshape(1, 3 * 16 * 16)
    return T_node, bias_node, G_pos, bias_pos


def _front_body(s_ref, p_ref, T_ref, bn_ref, G_ref, bp_ref, F_ref, bf_ref,
                wl_ref, bl_ref, wr_ref, br_ref, xl_ref, xr_ref, pos_ref):
    s = s_ref[...]
    out_node = jnp.dot(s, T_ref[...], preferred_element_type=jnp.float32)
    out_node = out_node + bn_ref[...]
    U = jnp.concatenate([p_ref[...], s[:, 56:64]], axis=1)
    out_pos = jnp.dot(U, G_ref[...], preferred_element_type=jnp.float32)
    out_pos = out_pos + bp_ref[...]

    def glu(o):
        k = o.shape[1] // 3
        return jnp.maximum(
            jnp.tanh(o[:, :k]) * jax.nn.sigmoid(o[:, k:2 * k]) + o[:, 2 * k:], 0.0)

    g_node = glu(out_node)
    g_pos = glu(out_pos)
    xl_ref[...] = jnp.dot(g_node, wl_ref[...],
                          preferred_element_type=jnp.float32) + bl_ref[...]
    xr_ref[...] = jnp.dot(g_node, wr_ref[...],
                          preferred_element_type=jnp.float32) + br_ref[...]
    pos_ref[...] = jnp.dot(g_pos, F_ref[...],
                           preferred_element_type=jnp.float32) + bf_ref[...]


def _mbuild_body(dst_ref, src_ref, m_ref):
    jt = lax.broadcasted_iota(jnp.int32, (N, E), 0)
    D = (jt == dst_ref[...]).astype(jnp.float32)
    it = lax.broadcasted_iota(jnp.int32, (E, N), 1)
    S = (it == src_ref[...]).astype(jnp.float32)
    m_ref[...] = jnp.dot(D, S, preferred_element_type=jnp.float32)


def _gat_body(xl_ref, xr_ref, m_ref, att_ref, w1_ref, w2m_ref, cv_ref, pre_ref):
    r = lax.broadcasted_iota(jnp.int32, (N, N), 0)
    c = lax.broadcasted_iota(jnp.int32, (N, N), 1)
    M = m_ref[...] + (r == c).astype(jnp.float32)
    mask = M > 0.0
    outs = []
    for h in range(H):
        Xl = xl_ref[0, :, h * C:(h + 1) * C]
        Xr = xr_ref[0, :, h * C:(h + 1) * C]
        att = att_ref[...][h][None, None, :]
        Z = Xl[None, :, :] + Xr[:, None, :]
        Z = jnp.where(Z >= 0.0, Z, 0.2 * Z)
        S = jnp.sum(Z * att, axis=-1)
        Sm = jnp.where(mask, S, -1e30)
        mx = jnp.max(Sm, axis=1, keepdims=True)
        P = jnp.exp(Sm - mx) * M
        den = jnp.sum(P, axis=1, keepdims=True)
        Aw = P / (den + 1e-16)
        outs.append(jnp.dot(Aw, Xl, preferred_element_type=jnp.float32))
    out = jnp.concatenate(outs, axis=1)
    q = jnp.sum(out * w1_ref[...], axis=0, keepdims=True)
    acc = jnp.dot(q, w2m_ref[...], preferred_element_type=jnp.float32)
    pre_ref[...] = jax.nn.sigmoid(acc + cv_ref[...])[None]


def kernel(data, position_embed, edge_index, ws_w, ws_b,
           d1w0, d1b0, d1w1, d1b1, d1w2, d1b2,
           d2w0, d2b0, d2w1, d2b1, d2w2, d2b2,
           d3w0, d3b0, d3w1, d3b1, d3w2, d3b2,
           fc3_w, fc3_b, gat_wl, gat_bl, gat_wr, gat_br, gat_att, gat_b,
           fc2_w, fc2_b, fc1_w, fc1_b):
    data = data.astype(jnp.float32)
    train = data[:, :62, :]
    target = data[:, 62:63, :]
    s = train.reshape(R, T)
    p = jnp.transpose(position_embed, (0, 2, 1, 3)).reshape(R, 32 * POS)

    convs = [(d1w0, d1b0, d1w1, d1b1, d1w2, d1b2),
             (d2w0, d2b0, d2w1, d2b1, d2w2, d2b2),
             (d3w0, d3b0, d3w1, d3b1, d3w2, d3b2)]
    T_node, bias_node, G_pos, bias_pos = _front_consts(ws_w, ws_b, convs)

    F = jnp.einsum('oc,wv->cwov', fc3_w, jnp.eye(POS, dtype=jnp.float32))
    F = F.reshape(16 * POS, 32 * POS)
    bias_f = jnp.repeat(fc3_b, POS)[None, :]

    xl, xr, pos_out = pl.pallas_call(
        _front_body,
        out_shape=[jax.ShapeDtypeStruct((R, H * C), jnp.float32),
                   jax.ShapeDtypeStruct((R, H * C), jnp.float32),
                   jax.ShapeDtypeStruct((R, 32 * POS), jnp.float32)],
    )(s, p, T_node, bias_node, G_pos, bias_pos, F, bias_f,
      gat_wl, gat_bl[None, :], gat_wr, gat_br[None, :])

    dst = edge_index[1].astype(jnp.int32)[None, :]
    src = edge_index[0].astype(jnp.int32)[:, None]
    Mcnt = pl.pallas_call(
        _mbuild_body,
        out_shape=jax.ShapeDtypeStruct((N, N), jnp.float32),
    )(dst, src)

    w1col = fc1_w[0][:, None]
    w1sum = jnp.sum(fc1_w)
    W2mat = jnp.einsum('k,tv->ktv', fc2_w[0],
                       jnp.eye(T, dtype=jnp.float32)).reshape(8 * T, T)
    gb = jnp.sum(gat_b.reshape(8, T) * fc2_w[0][:, None], axis=0)
    constv = (fc1_b[0] + w1sum * fc2_b[0] + w1sum * gb)[None, :]

    pre = pl.pallas_call(
        _gat_body,
        grid=(B,),
        in_specs=[
            pl.BlockSpec((1, N, H * C), lambda b: (b, 0, 0)),
            pl.BlockSpec((1, N, H * C), lambda b: (b, 0, 0)),
            pl.BlockSpec((N, N), lambda b: (0, 0)),
            pl.BlockSpec((H, C), lambda b: (0, 0)),
            pl.BlockSpec((N, 1), lambda b: (0, 0)),
            pl.BlockSpec((8 * T, T), lambda b: (0, 0)),
            pl.BlockSpec((1, T), lambda b: (0, 0)),
        ],
        out_specs=pl.BlockSpec((1, 1, T), lambda b: (b, 0, 0)),
        out_shape=jax.ShapeDtypeStruct((B, 1, T), jnp.float32),
    )(xl.reshape(B, N, H * C), xr.reshape(B, N, H * C), Mcnt,
      gat_att, w1col, W2mat, constv)

    pos_learned = jnp.transpose(pos_out.reshape(B, N, 32, POS), (0, 2, 1, 3))
    return (pre, target, pos_learned)
